# Initial kernel scaffold; baseline (speedup 1.0000x reference)
import jax
import jax.numpy as jnp
from jax.experimental import pallas as pl


def kernel(tokens, emb_table):
    raise NotImplementedError("write your pallas kernel here")


# speedup vs baseline: 3.6170x; 3.6170x over previous
import functools

import jax
import jax.numpy as jnp
from jax import lax
from jax.experimental import pallas as pl
from jax.experimental.pallas import tpu as pltpu
from jax.experimental.pallas import tpu_sc as plsc

SEQ = 200
EMB = 64
VOC = 100
BATCH = 4096
SCALE = float(EMB) ** 0.5

NW = 32
TOK_TOTAL = BATCH * SEQ
TOK_W = TOK_TOTAL // NW
CH = 128
NCH = TOK_W // CH


def _pos_table():
    pos = jnp.arange(SEQ, dtype=jnp.float32)[:, None]
    div = 10000.0 ** (jnp.arange(0, EMB, 2, dtype=jnp.float32) / EMB)
    pe = jnp.zeros((SEQ, EMB), dtype=jnp.float32)
    pe = pe.at[:, 0::2].set(jnp.sin(pos / div))
    pe = pe.at[:, 1::2].set(jnp.cos(pos / div))
    return pe


def _fused_table(emb_table, pos):
    def body(t_ref, p_ref, o_ref):
        t = t_ref[...] * SCALE
        p = p_ref[...]
        f = p[:, None, :] + t[None, :, :]
        o_ref[...] = f.reshape(SEQ * VOC, EMB)

    return pl.pallas_call(
        body,
        out_shape=jax.ShapeDtypeStruct((SEQ * VOC, EMB), jnp.float32),
    )(emb_table, pos)


_mesh = plsc.VectorSubcoreMesh(core_axis_name="c", subcore_axis_name="s")


@functools.partial(
    pl.kernel,
    mesh=_mesh,
    compiler_params=pltpu.CompilerParams(use_tc_tiling_on_sc=False),
    out_type=jax.ShapeDtypeStruct((TOK_TOTAL, EMB), jnp.float32),
    scratch_types=[
        pltpu.VMEM((TOK_W,), jnp.int32),
        pltpu.VMEM((CH,), jnp.int32),
        pltpu.VMEM((CH, EMB), jnp.float32),
        pltpu.SemaphoreType.DMA,
    ],
)
def _sc_gather(tok_hbm, fused_hbm, out_hbm, tok_v, idx_v, rows_v, gsem):
    wid = lax.axis_index("s") * 2 + lax.axis_index("c")
    tbase = wid * TOK_W
    pltpu.sync_copy(tok_hbm.at[pl.ds(tbase, TOK_W)], tok_v)
    lane = lax.iota(jnp.int32, 16)

    def chunk_body(c, carry):
        def vec_body(i, carry2):
            off = c * CH + i * 16
            base = lax.rem(off, SEQ)
            p = base + lane
            s = jnp.where(p >= SEQ, p - SEQ, p)
            tk = tok_v[pl.ds(off, 16)]
            idx_v[pl.ds(i * 16, 16)] = s * VOC + tk
            return carry2

        lax.fori_loop(0, CH // 16, vec_body, 0)
        pltpu.async_copy(fused_hbm.at[idx_v], rows_v, gsem).wait()
        pltpu.sync_copy(rows_v, out_hbm.at[pl.ds(tbase + c * CH, CH)])
        return carry

    lax.fori_loop(0, NCH, chunk_body, 0)


def kernel(tokens, emb_table):
    pos = _pos_table()
    fused = _fused_table(emb_table, pos)
    tok_flat = tokens.reshape(TOK_TOTAL).astype(jnp.int32)
    out = _sc_gather(tok_flat, fused)
    return out.reshape(BATCH, SEQ, EMB)


# speedup vs baseline: 4.4106x; 1.2194x over previous
import functools

import jax
import jax.numpy as jnp
from jax import lax
from jax.experimental import pallas as pl
from jax.experimental.pallas import tpu as pltpu
from jax.experimental.pallas import tpu_sc as plsc

SEQ = 200
EMB = 64
VOC = 100
BATCH = 4096
SCALE = float(EMB) ** 0.5

NW = 32
TOK_TOTAL = BATCH * SEQ
TOK_W = TOK_TOTAL // NW
CH = 128
GN = 4
GCH = CH * GN
NG = TOK_W // GCH


def _pos_table():
    pos = jnp.arange(SEQ, dtype=jnp.float32)[:, None]
    div = 10000.0 ** (jnp.arange(0, EMB, 2, dtype=jnp.float32) / EMB)
    pe = jnp.zeros((SEQ, EMB), dtype=jnp.float32)
    pe = pe.at[:, 0::2].set(jnp.sin(pos / div))
    pe = pe.at[:, 1::2].set(jnp.cos(pos / div))
    return pe


def _fused_table(emb_table, pos):
    def body(t_ref, p_ref, o_ref):
        t = t_ref[...] * SCALE
        p = p_ref[...]
        f = p[:, None, :] + t[None, :, :]
        o_ref[...] = f.reshape(SEQ * VOC, EMB)

    return pl.pallas_call(
        body,
        out_shape=jax.ShapeDtypeStruct((SEQ * VOC, EMB), jnp.float32),
    )(emb_table, pos)


_mesh = plsc.VectorSubcoreMesh(core_axis_name="c", subcore_axis_name="s")


@functools.partial(
    pl.kernel,
    mesh=_mesh,
    compiler_params=pltpu.CompilerParams(use_tc_tiling_on_sc=False),
    out_type=jax.ShapeDtypeStruct((TOK_TOTAL, EMB), jnp.float32),
    scratch_types=[
        pltpu.VMEM((TOK_W,), jnp.int32),
        pltpu.VMEM((GN, CH), jnp.int32),
        pltpu.VMEM((GN, CH), jnp.int32),
        pltpu.VMEM((GCH, EMB), jnp.float32),
        pltpu.VMEM((GCH, EMB), jnp.float32),
        pltpu.SemaphoreType.DMA,
        pltpu.SemaphoreType.DMA,
    ],
)
def _sc_gather(tok_hbm, fused_hbm, out_hbm, tok_v, idx_a, idx_b, rows_a,
               rows_b, gsem_a, gsem_b):
    wid = lax.axis_index("s") * 2 + lax.axis_index("c")
    tbase = wid * TOK_W
    pltpu.sync_copy(tok_hbm.at[pl.ds(tbase, TOK_W)], tok_v)
    lane = lax.iota(jnp.int32, 16)

    def compute_idx(idx_ref, g):
        for j in range(GN):
            def vec_body(i, carry, j=j):
                off = g * GCH + j * CH + i * 16
                base = lax.rem(off, SEQ)
                p = base + lane
                s = jnp.where(p >= SEQ, p - SEQ, p)
                tk = tok_v[pl.ds(off, 16)]
                idx_ref[j, pl.ds(i * 16, 16)] = s * VOC + tk
                return carry

            lax.fori_loop(0, CH // 16, vec_body, 0)

    def fire_gathers(idx_ref, rows_ref, sem):
        for j in range(GN):
            pltpu.async_copy(fused_hbm.at[idx_ref.at[j]],
                             rows_ref.at[pl.ds(j * CH, CH)], sem)

    def drain_gathers(rows_ref, sem):
        pltpu.make_async_copy(out_hbm.at[pl.ds(0, GCH)], rows_ref, sem).wait()

    def scatter(rows_ref, g):
        pltpu.sync_copy(rows_ref, out_hbm.at[pl.ds(tbase + g * GCH, GCH)])

    compute_idx(idx_a, 0)
    fire_gathers(idx_a, rows_a, gsem_a)

    def body(i, carry):
        g_a = 2 * i
        g_b = 2 * i + 1
        compute_idx(idx_b, g_b)
        fire_gathers(idx_b, rows_b, gsem_b)
        drain_gathers(rows_a, gsem_a)
        scatter(rows_a, g_a)

        g_a2 = 2 * i + 2
        @pl.when(g_a2 < NG)
        def _():
            compute_idx(idx_a, g_a2)
            fire_gathers(idx_a, rows_a, gsem_a)

        drain_gathers(rows_b, gsem_b)
        scatter(rows_b, g_b)
        return carry

    lax.fori_loop(0, NG // 2, body, 0)


def kernel(tokens, emb_table):
    pos = _pos_table()
    fused = _fused_table(emb_table, pos)
    tok_flat = tokens.reshape(TOK_TOTAL).astype(jnp.int32)
    out = _sc_gather(tok_flat, fused)
    return out.reshape(BATCH, SEQ, EMB)


# speedup vs baseline: 5.5236x; 1.2523x over previous
import functools

import jax
import jax.numpy as jnp
from jax import lax
from jax.experimental import pallas as pl
from jax.experimental.pallas import tpu as pltpu
from jax.experimental.pallas import tpu_sc as plsc

SEQ = 200
EMB = 64
VOC = 100
BATCH = 4096
SCALE = float(EMB) ** 0.5

NW = 32
BT = BATCH // NW
PAD = 128


def _pos_table():
    pos = jnp.arange(SEQ, dtype=jnp.float32)[:, None]
    div = 10000.0 ** (jnp.arange(0, EMB, 2, dtype=jnp.float32) / EMB)
    pe = jnp.zeros((SEQ, EMB), dtype=jnp.float32)
    pe = pe.at[:, 0::2].set(jnp.sin(pos / div))
    pe = pe.at[:, 1::2].set(jnp.cos(pos / div))
    return pe


def _fused_table(emb_table, pos):
    def body(t_ref, p_ref, o_ref):
        t = t_ref[...] * SCALE
        p = p_ref[...]
        f = p[:, None, :] + t[None, :, :]
        o_ref[...] = jnp.pad(f.reshape(SEQ * VOC, EMB),
                             ((0, 0), (0, PAD - EMB)))

    return pl.pallas_call(
        body,
        out_shape=jax.ShapeDtypeStruct((SEQ * VOC, PAD), jnp.float32),
    )(emb_table, pos)


_mesh = plsc.VectorSubcoreMesh(core_axis_name="c", subcore_axis_name="s")


@functools.partial(
    pl.kernel,
    mesh=_mesh,
    compiler_params=pltpu.CompilerParams(needs_layout_passes=False),
    out_type=jax.ShapeDtypeStruct((SEQ, EMB, BATCH), jnp.float32),
    scratch_types=[
        pltpu.VMEM((BT, SEQ), jnp.int32),
        pltpu.VMEM((BT,), jnp.int32),
        pltpu.VMEM((BT,), jnp.int32),
        pltpu.VMEM((BT, PAD), jnp.float32),
        pltpu.VMEM((BT, PAD), jnp.float32),
        pltpu.VMEM((EMB, BT), jnp.float32),
        pltpu.SemaphoreType.DMA,
        pltpu.SemaphoreType.DMA,
    ],
)
def _sc_embed(tok_hbm, fused_hbm, out_hbm, tok_v, idx_a, idx_b, rows_a,
              rows_b, trans_v, gsem_a, gsem_b):
    wid = lax.axis_index("s") * 2 + lax.axis_index("c")
    bbase = wid * BT
    pltpu.sync_copy(tok_hbm.at[pl.ds(bbase, BT)], tok_v)
    lane = lax.iota(jnp.int32, 16)

    def compute_idx(idx_ref, s):
        for i in range(BT // 16):
            r = i * 16 + lane
            tk = plsc.load_gather(tok_v, [r, jnp.zeros((16,), jnp.int32) + s])
            idx_ref[pl.ds(i * 16, 16)] = s * VOC + tk

    def fire_gather(idx_ref, rows_ref, sem):
        pltpu.async_copy(fused_hbm.at[idx_ref], rows_ref, sem)

    def drain_gather(rows_ref, sem):
        pltpu.make_async_copy(fused_hbm.at[pl.ds(0, BT)], rows_ref,
                              sem).wait()

    def emit_block(rows_ref, s):
        def kbody(k, carry):
            rot = jnp.bitwise_and(lane + k, 15)
            for t0 in range(0, BT, 16):
                rvec = t0 + lane
                for e0 in range(0, EMB, 16):
                    cvec = e0 + rot
                    v = plsc.load_gather(rows_ref, [rvec, cvec])
                    plsc.store_scatter(trans_v, [cvec, rvec], v)
            return carry

        lax.fori_loop(0, 16, kbody, 0)
        pltpu.sync_copy(trans_v, out_hbm.at[s, :, pl.ds(bbase, BT)])

    compute_idx(idx_a, 0)
    fire_gather(idx_a, rows_a, gsem_a)

    def body(i, carry):
        s_a = 2 * i
        s_b = 2 * i + 1
        compute_idx(idx_b, s_b)
        fire_gather(idx_b, rows_b, gsem_b)
        drain_gather(rows_a, gsem_a)
        emit_block(rows_a, s_a)

        s_a2 = 2 * i + 2
        @pl.when(s_a2 < SEQ)
        def _():
            compute_idx(idx_a, s_a2)
            fire_gather(idx_a, rows_a, gsem_a)

        drain_gather(rows_b, gsem_b)
        emit_block(rows_b, s_b)
        return carry

    lax.fori_loop(0, SEQ // 2, body, 0)


def kernel(tokens, emb_table):
    pos = _pos_table()
    fused = _fused_table(emb_table, pos)
    out_t = _sc_embed(tokens.astype(jnp.int32), fused)
    return out_t.transpose(2, 0, 1)


# speedup vs baseline: 6.1131x; 1.1067x over previous
import functools

import jax
import jax.numpy as jnp
from jax import lax
from jax.experimental import pallas as pl
from jax.experimental.pallas import tpu as pltpu
from jax.experimental.pallas import tpu_sc as plsc

SEQ = 200
EMB = 64
VOC = 100
BATCH = 4096
SCALE = float(EMB) ** 0.5

NW = 32
BT = BATCH // NW
PAD = 128


def _pos_table():
    pos = jnp.arange(SEQ, dtype=jnp.float32)[:, None]
    div = 10000.0 ** (jnp.arange(0, EMB, 2, dtype=jnp.float32) / EMB)
    pe = jnp.zeros((SEQ, EMB), dtype=jnp.float32)
    pe = pe.at[:, 0::2].set(jnp.sin(pos / div))
    pe = pe.at[:, 1::2].set(jnp.cos(pos / div))
    return pe


def _fused_table(emb_table, pos):
    def body(t_ref, p_ref, o_ref):
        t = t_ref[...] * SCALE
        p = p_ref[...]
        f = p[:, None, :] + t[None, :, :]
        o_ref[...] = jnp.pad(f.reshape(SEQ * VOC, EMB),
                             ((0, 0), (0, PAD - EMB)))

    return pl.pallas_call(
        body,
        out_shape=jax.ShapeDtypeStruct((SEQ * VOC, PAD), jnp.float32),
    )(emb_table, pos)


_mesh = plsc.VectorSubcoreMesh(core_axis_name="c", subcore_axis_name="s")


@functools.partial(
    pl.kernel,
    mesh=_mesh,
    compiler_params=pltpu.CompilerParams(needs_layout_passes=False),
    out_type=jax.ShapeDtypeStruct((SEQ, EMB, BATCH), jnp.float32),
    scratch_types=[
        pltpu.VMEM((BT, SEQ), jnp.int32),
        pltpu.VMEM((BT,), jnp.int32),
        pltpu.VMEM((BT,), jnp.int32),
        pltpu.VMEM((BT, PAD), jnp.float32),
        pltpu.VMEM((BT, PAD), jnp.float32),
        pltpu.VMEM((EMB, BT), jnp.float32),
        pltpu.VMEM((EMB, BT), jnp.float32),
        pltpu.SemaphoreType.DMA,
        pltpu.SemaphoreType.DMA,
        pltpu.SemaphoreType.DMA,
        pltpu.SemaphoreType.DMA,
    ],
)
def _sc_embed(tok_hbm, fused_hbm, out_hbm, tok_v, idx_a, idx_b, rows_a,
              rows_b, trans_a, trans_b, gsem_a, gsem_b, ssem_a, ssem_b):
    wid = lax.axis_index("s") * 2 + lax.axis_index("c")
    bbase = wid * BT
    pltpu.sync_copy(tok_hbm.at[pl.ds(bbase, BT)], tok_v)
    lane = lax.iota(jnp.int32, 16)

    def compute_idx(idx_ref, s):
        for i in range(BT // 16):
            r = i * 16 + lane
            tk = plsc.load_gather(tok_v, [r, jnp.zeros((16,), jnp.int32) + s])
            idx_ref[pl.ds(i * 16, 16)] = s * VOC + tk

    def fire_gather(idx_ref, rows_ref, sem):
        pltpu.async_copy(fused_hbm.at[idx_ref], rows_ref, sem)

    def drain_gather(rows_ref, sem):
        pltpu.make_async_copy(fused_hbm.at[pl.ds(0, BT)], rows_ref,
                              sem).wait()

    def emit_block(rows_ref, trans_ref, ssem, s):
        @pl.when(s >= 2)
        def _():
            pltpu.make_async_copy(trans_ref,
                                  out_hbm.at[0, :, pl.ds(bbase, BT)],
                                  ssem).wait()

        def kbody(k, carry):
            rot = jnp.bitwise_and(lane + k, 15)
            for t0 in range(0, BT, 16):
                rvec = t0 + lane
                for e0 in range(0, EMB, 16):
                    cvec = e0 + rot
                    v = plsc.load_gather(rows_ref, [rvec, cvec])
                    plsc.store_scatter(trans_ref, [cvec, rvec], v)
            return carry

        lax.fori_loop(0, 16, kbody, 0)
        pltpu.async_copy(trans_ref, out_hbm.at[s, :, pl.ds(bbase, BT)], ssem)

    compute_idx(idx_a, 0)
    fire_gather(idx_a, rows_a, gsem_a)

    def body(i, carry):
        s_a = 2 * i
        s_b = 2 * i + 1
        compute_idx(idx_b, s_b)
        fire_gather(idx_b, rows_b, gsem_b)
        drain_gather(rows_a, gsem_a)
        emit_block(rows_a, trans_a, ssem_a, s_a)

        s_a2 = 2 * i + 2
        @pl.when(s_a2 < SEQ)
        def _():
            compute_idx(idx_a, s_a2)
            fire_gather(idx_a, rows_a, gsem_a)

        drain_gather(rows_b, gsem_b)
        emit_block(rows_b, trans_b, ssem_b, s_b)
        return carry

    lax.fori_loop(0, SEQ // 2, body, 0)

    pltpu.make_async_copy(trans_a, out_hbm.at[0, :, pl.ds(bbase, BT)],
                          ssem_a).wait()
    pltpu.make_async_copy(trans_b, out_hbm.at[0, :, pl.ds(bbase, BT)],
                          ssem_b).wait()


def kernel(tokens, emb_table):
    pos = _pos_table()
    fused = _fused_table(emb_table, pos)
    out_t = _sc_embed(tokens.astype(jnp.int32), fused)
    return out_t.transpose(2, 0, 1)


# speedup vs baseline: 6.2835x; 1.0279x over previous
import functools

import jax
import jax.numpy as jnp
from jax import lax
from jax.experimental import pallas as pl
from jax.experimental.pallas import tpu as pltpu
from jax.experimental.pallas import tpu_sc as plsc

SEQ = 200
EMB = 64
VOC = 100
BATCH = 4096
SCALE = float(EMB) ** 0.5

NW = 32
BT = BATCH // NW
PAD = 128


def _pos_table():
    pos = jnp.arange(SEQ, dtype=jnp.float32)[:, None]
    div = 10000.0 ** (jnp.arange(0, EMB, 2, dtype=jnp.float32) / EMB)
    pe = jnp.zeros((SEQ, EMB), dtype=jnp.float32)
    pe = pe.at[:, 0::2].set(jnp.sin(pos / div))
    pe = pe.at[:, 1::2].set(jnp.cos(pos / div))
    return pe


def _fused_table(emb_table, pos):
    def body(t_ref, p_ref, o_ref):
        t = t_ref[...] * SCALE
        p = p_ref[...]
        f = p[:, None, :] + t[None, :, :]
        o_ref[...] = jnp.pad(f.reshape(SEQ * VOC, EMB),
                             ((0, 0), (0, PAD - EMB)))

    return pl.pallas_call(
        body,
        out_shape=jax.ShapeDtypeStruct((SEQ * VOC, PAD), jnp.float32),
    )(emb_table, pos)


_mesh = plsc.VectorSubcoreMesh(core_axis_name="c", subcore_axis_name="s")


@functools.partial(
    pl.kernel,
    mesh=_mesh,
    compiler_params=pltpu.CompilerParams(needs_layout_passes=False),
    out_type=jax.ShapeDtypeStruct((SEQ, EMB, BATCH), jnp.float32),
    scratch_types=[
        pltpu.VMEM((BT, SEQ), jnp.int32),
        pltpu.VMEM((4, BT), jnp.int32),
        pltpu.VMEM((BT, PAD), jnp.float32),
        pltpu.VMEM((BT, PAD), jnp.float32),
        pltpu.VMEM((BT, PAD), jnp.float32),
        pltpu.VMEM((BT, PAD), jnp.float32),
        pltpu.VMEM((EMB, BT), jnp.float32),
        pltpu.VMEM((EMB, BT), jnp.float32),
        pltpu.SemaphoreType.DMA,
        pltpu.SemaphoreType.DMA,
        pltpu.SemaphoreType.DMA,
        pltpu.SemaphoreType.DMA,
        pltpu.SemaphoreType.DMA,
        pltpu.SemaphoreType.DMA,
    ],
)
def _sc_embed(tok_hbm, fused_hbm, out_hbm, tok_v, idx_v, rows0, rows1,
              rows2, rows3, trans_a, trans_b, gsem0, gsem1, gsem2, gsem3,
              ssem_a, ssem_b):
    wid = lax.axis_index("s") * 2 + lax.axis_index("c")
    bbase = wid * BT
    pltpu.sync_copy(tok_hbm.at[pl.ds(bbase, BT)], tok_v)
    lane = lax.iota(jnp.int32, 16)

    def compute_idx(idx_ref, s):
        for i in range(BT // 16):
            r = i * 16 + lane
            tk = plsc.load_gather(tok_v, [r, jnp.zeros((16,), jnp.int32) + s])
            idx_ref[pl.ds(i * 16, 16)] = s * VOC + tk

    def fire_gather(idx_ref, rows_ref, sem):
        pltpu.async_copy(fused_hbm.at[idx_ref], rows_ref, sem)

    def drain_gather(rows_ref, sem):
        pltpu.make_async_copy(fused_hbm.at[pl.ds(0, BT)], rows_ref,
                              sem).wait()

    def emit_block(rows_ref, trans_ref, ssem, s):
        @pl.when(s >= 2)
        def _():
            pltpu.make_async_copy(trans_ref,
                                  out_hbm.at[0, :, pl.ds(bbase, BT)],
                                  ssem).wait()

        def kbody(k, carry):
            rot = jnp.bitwise_and(lane + k, 15)
            for t0 in range(0, BT, 16):
                rvec = t0 + lane
                for e0 in range(0, EMB, 16):
                    cvec = e0 + rot
                    v = plsc.load_gather(rows_ref, [rvec, cvec])
                    plsc.store_scatter(trans_ref, [cvec, rvec], v)
            return carry

        lax.fori_loop(0, 16, kbody, 0)
        pltpu.async_copy(trans_ref, out_hbm.at[s, :, pl.ds(bbase, BT)], ssem)

    rows = [rows0, rows1, rows2, rows3]
    gsem = [gsem0, gsem1, gsem2, gsem3]
    trans = [trans_a, trans_b]
    ssem = [ssem_a, ssem_b]

    for j in range(4):
        compute_idx(idx_v.at[j], j)
        fire_gather(idx_v.at[j], rows[j], gsem[j])

    def body(i, carry):
        for j in range(4):
            s = 4 * i + j
            drain_gather(rows[j], gsem[j])
            emit_block(rows[j], trans[j & 1], ssem[j & 1], s)

            @pl.when(s + 4 < SEQ)
            def _(j=j, s=s):
                compute_idx(idx_v.at[j], s + 4)
                fire_gather(idx_v.at[j], rows[j], gsem[j])
        return carry

    lax.fori_loop(0, SEQ // 4, body, 0)

    pltpu.make_async_copy(trans_a, out_hbm.at[0, :, pl.ds(bbase, BT)],
                          ssem_a).wait()
    pltpu.make_async_copy(trans_b, out_hbm.at[0, :, pl.ds(bbase, BT)],
                          ssem_b).wait()


def kernel(tokens, emb_table):
    pos = _pos_table()
    fused = _fused_table(emb_table, pos)
    out_t = _sc_embed(tokens.astype(jnp.int32), fused)
    return out_t.transpose(2, 0, 1)
